# Initial kernel scaffold; baseline (speedup 1.0000x reference)
import jax
import jax.numpy as jnp
from jax.experimental import pallas as pl


def kernel(indices, table_one, table_two):
    raise NotImplementedError("write your pallas kernel here")


# speedup vs baseline: 2.3756x; 2.3756x over previous
import functools

import jax
import jax.numpy as jnp
from jax import lax
from jax.experimental import pallas as pl
from jax.experimental.pallas import tpu as pltpu
from jax.experimental.pallas import tpu_sc as plsc

V1 = 100000
D = 64
B = 16384
NC = 2
NS = 16
NW = NC * NS
BPW = B // NW
GCH = 128
NG = BPW // GCH
L = 16


def _body(idx_hbm, t1_hbm, t2_hbm, out_hbm,
          idx_v, idx1_v, idx2_v, sel_v, buf1, buf2, sem):
    wid = lax.axis_index("s") * NC + lax.axis_index("c")
    base = wid * BPW

    pltpu.sync_copy(idx_hbm.at[pl.ds(base, BPW)], idx_v)

    zero = jnp.zeros((L,), jnp.float32)
    for c in range(D // L):
        buf2[BPW, pl.ds(c * L, L)] = zero

    iota = lax.broadcasted_iota(jnp.int32, (L,), 0)
    for c in range(BPW // L):
        v = idx_v[pl.ds(c * L, L)]
        is2 = v >= V1
        idx1 = jnp.where(is2, 0, v)
        idx2 = jnp.where(is2, v - (V1 - 1), 0)
        sel = jnp.where(is2, iota + c * L, BPW)
        idx1_v[c // (GCH // L), pl.ds((c % (GCH // L)) * L, L)] = idx1
        idx2_v[c // (GCH // L), pl.ds((c % (GCH // L)) * L, L)] = idx2
        sel_v[pl.ds(c * L, L)] = sel

    copies = []
    for k in range(NG):
        copies.append(pltpu.async_copy(
            t1_hbm.at[idx1_v.at[k]], buf1.at[pl.ds(k * GCH, GCH)], sem))
        copies.append(pltpu.async_copy(
            t2_hbm.at[idx2_v.at[k]], buf2.at[pl.ds(k * GCH, GCH)], sem))
    for cp in copies:
        cp.wait()

    def combine(ch, carry):
        sv = sel_v[pl.ds(ch * L, L)]
        for j in range(L):
            s = sv[j]
            r = ch * L + j
            for c in range(D // L):
                sl = pl.ds(c * L, L)
                plsc.addupdate(buf1.at[r, sl], buf2[s, sl])
        return carry

    lax.fori_loop(0, BPW // L, combine, 0)

    pltpu.sync_copy(buf1, out_hbm.at[pl.ds(base, BPW)])


@jax.jit
def _split_embedding(indices, table_one, table_two):
    mesh = plsc.VectorSubcoreMesh(
        core_axis_name="c", subcore_axis_name="s",
        num_cores=NC, num_subcores=NS)
    return pl.kernel(
        _body,
        out_type=jax.ShapeDtypeStruct((B, D), jnp.float32),
        mesh=mesh,
        compiler_params=pltpu.CompilerParams(use_tc_tiling_on_sc=False),
        scratch_types=[
            pltpu.VMEM((BPW,), jnp.int32),
            pltpu.VMEM((NG, GCH), jnp.int32),
            pltpu.VMEM((NG, GCH), jnp.int32),
            pltpu.VMEM((BPW,), jnp.int32),
            pltpu.VMEM((BPW, D), jnp.float32),
            pltpu.VMEM((BPW + 1, D), jnp.float32),
            pltpu.SemaphoreType.DMA,
        ],
    )(indices, table_one, table_two)


def kernel(indices, table_one, table_two):
    return _split_embedding(indices, table_one, table_two)


# speedup vs baseline: 4.6943x; 1.9761x over previous
import jax
import jax.numpy as jnp
from jax import lax
from jax.experimental import pallas as pl
from jax.experimental.pallas import tpu as pltpu
from jax.experimental.pallas import tpu_sc as plsc

V1 = 100000
D = 64
B = 16384
NC = 2
NS = 16
NW = NC * NS
BPW = B // NW
GCH = 128
NG = BPW // GCH
L = 16


def _body(idx_hbm, t1_hbm, t2_hbm, out_hbm, idx_v, idxm_v, sel_v, buf, sem):
    wid = lax.axis_index("s") * NC + lax.axis_index("c")
    base = wid * BPW

    pltpu.sync_copy(idx_hbm.at[pl.ds(base, BPW)], idx_v)

    iota = lax.broadcasted_iota(jnp.int32, (L,), 0)
    for c in range(BPW // L):
        v = idx_v[pl.ds(c * L, L)]
        is2 = v >= V1
        idxm = jnp.where(is2, v - (V1 - 1), v)
        sel = jnp.where(is2, iota + (BPW + c * L), iota + c * L)
        idxm_v[c // (GCH // L), pl.ds((c % (GCH // L)) * L, L)] = idxm
        sel_v[pl.ds(c * L, L)] = sel

    copies = []
    for k in range(NG):
        copies.append(pltpu.async_copy(
            t1_hbm.at[idxm_v.at[k]], buf.at[pl.ds(k * GCH, GCH)], sem))
        copies.append(pltpu.async_copy(
            t2_hbm.at[idxm_v.at[k]], buf.at[pl.ds(BPW + k * GCH, GCH)], sem))
    for cp in copies:
        cp.wait()

    def combine(ch, carry):
        sv = sel_v[pl.ds(ch * L, L)]
        for j in range(L):
            s = sv[j]
            r = ch * L + j
            for c in range(D // L):
                sl = pl.ds(c * L, L)
                buf[r, sl] = buf[s, sl]
        return carry

    lax.fori_loop(0, BPW // L, combine, 0)

    pltpu.sync_copy(buf.at[pl.ds(0, BPW)], out_hbm.at[pl.ds(base, BPW)])


@jax.jit
def _split_embedding(indices, table_one, table_two):
    mesh = plsc.VectorSubcoreMesh(
        core_axis_name="c", subcore_axis_name="s",
        num_cores=NC, num_subcores=NS)
    return pl.kernel(
        _body,
        out_type=jax.ShapeDtypeStruct((B, D), jnp.float32),
        mesh=mesh,
        compiler_params=pltpu.CompilerParams(use_tc_tiling_on_sc=False),
        scratch_types=[
            pltpu.VMEM((BPW,), jnp.int32),
            pltpu.VMEM((NG, GCH), jnp.int32),
            pltpu.VMEM((BPW,), jnp.int32),
            pltpu.VMEM((2 * BPW, D), jnp.float32),
            pltpu.SemaphoreType.DMA,
        ],
    )(indices, table_one, table_two)


def kernel(indices, table_one, table_two):
    return _split_embedding(indices, table_one, table_two)
